# baseline (speedup 1.0000x reference)
import functools

import jax
import jax.numpy as jnp
from jax import lax
from jax.experimental import pallas as pl
from jax.experimental.pallas import tpu as pltpu
from jax.experimental.pallas import tpu_sc as plsc

_EPS = 1e-8
_LANES = 16

_LOG1P_C = (
    0.9999999978515015, -0.49999970549016043, 0.3333233204385284,
    -0.24985166951419607, 0.1988030687513739, -0.1607517519300022,
    0.12363706209802593, -0.0818373380867524, 0.041082179468388526,
    -0.013268948579038478, 0.0020109670905085796,
)


def _silu(x):
    return x * (0.5 * jnp.tanh(0.5 * x) + 0.5)


def _tc1_body(a_ref, wq_ref, wk_ref, bq_ref, g0_ref, g1_ref):
    mt = lax.dot_general(wk_ref[...], wq_ref[...], (((1,), (1,)), ((), ())),
                         preferred_element_type=jnp.float32)
    bk = lax.dot_general(wk_ref[...], bq_ref[...], (((1,), (1,)), ((), ())),
                         preferred_element_type=jnp.float32)
    gt = lax.dot_general(mt, a_ref[...], (((1,), (1,)), ((), ())),
                         preferred_element_type=jnp.float32) + bk
    t = gt.shape[1]
    g0_ref[...] = gt[0:1, :].reshape(t)
    g1_ref[...] = gt[1:2, :].reshape(t)


def _silu_bf(x):
    xb = x.astype(jnp.bfloat16)
    return xb * (0.5 * jnp.tanh(0.5 * xb) + 0.5)


def _tc2_body(u0_ref, u1_ref, wv_ref, w1_ref, w2_ref, wo_ref, o_ref):
    bf = jnp.bfloat16
    t = u0_ref.shape[0]
    ut = jnp.concatenate([u0_ref[...].reshape(1, t),
                          u1_ref[...].reshape(1, t)], axis=0)
    x = lax.dot_general(ut, wv_ref[...], (((0,), (0,)), ((), ())),
                        preferred_element_type=jnp.float32)
    s = jnp.dot(_silu_bf(x), w1_ref[...].astype(bf),
                preferred_element_type=jnp.float32)
    y = x + jnp.dot(_silu_bf(s), w2_ref[...].astype(bf),
                    preferred_element_type=jnp.float32)
    o_ref[...] = jnp.dot(_silu_bf(y), wo_ref[...].astype(bf),
                         preferred_element_type=jnp.float32)


def _softplus16(d):
    t = jnp.exp(-jnp.abs(d))
    p = jnp.full((_LANES,), _LOG1P_C[-1], jnp.float32)
    for c in reversed(_LOG1P_C[:-1]):
        p = p * t + c
    return jnp.maximum(d, 0.0) + p * t


def _worker_extent(n, nw):
    chunk = ((n + nw - 1) // nw + _LANES - 1) // _LANES * _LANES
    last = n - (nw - 1) * chunk
    assert last > 0 and last % _LANES == 0 and chunk % _LANES == 0
    return chunk, last


def _make_sc1(n, b, nw, nc, inv_sqrt_d):
    chunk, last = _worker_extent(n, nw)
    mesh = plsc.VectorSubcoreMesh(core_axis_name="c", subcore_axis_name="s")

    @functools.partial(
        pl.kernel,
        mesh=mesh,
        compiler_params=pltpu.CompilerParams(needs_layout_passes=False),
        out_type=[
            jax.ShapeDtypeStruct((n,), jnp.float32),
            jax.ShapeDtypeStruct((nw, b), jnp.float32),
        ],
        scratch_types=[
            pltpu.VMEM((b,), jnp.float32),
            pltpu.VMEM((b,), jnp.float32),
            pltpu.VMEM((b,), jnp.float32),
            pltpu.VMEM((chunk,), jnp.int32),
            pltpu.VMEM((chunk,), jnp.float32),
            pltpu.VMEM((chunk,), jnp.float32),
            pltpu.VMEM((chunk,), jnp.float32),
            pltpu.VMEM((_LANES, b), jnp.float32),
            pltpu.VMEM((b,), jnp.float32),
        ],
    )
    def sc1(q_hbm, seg_hbm, g0_hbm, g1_hbm, a_hbm, part_hbm,
            q_v, t0_v, t1_v, seg_v, g0_v, g1_v, a_v, an16_v, part_v):
        wid = lax.axis_index("s") * nc + lax.axis_index("c")
        base = wid * chunk
        is_last = wid == nw - 1
        pltpu.sync_copy(q_hbm, q_v)

        @pl.when(jnp.logical_not(is_last))
        def _():
            pltpu.sync_copy(seg_hbm.at[pl.ds(base, chunk)], seg_v)
            pltpu.sync_copy(g0_hbm.at[pl.ds(base, chunk)], g0_v)
            pltpu.sync_copy(g1_hbm.at[pl.ds(base, chunk)], g1_v)

        @pl.when(is_last)
        def _():
            pltpu.sync_copy(seg_hbm.at[pl.ds(base, last)],
                            seg_v.at[pl.ds(0, last)])
            pltpu.sync_copy(g0_hbm.at[pl.ds(base, last)],
                            g0_v.at[pl.ds(0, last)])
            pltpu.sync_copy(g1_hbm.at[pl.ds(base, last)],
                            g1_v.at[pl.ds(0, last)])

        iota = lax.iota(jnp.int32, _LANES)
        zeros = jnp.zeros((_LANES,), jnp.float32)

        @plsc.parallel_loop(0, b // _LANES, unroll=2)
        def _(j):
            q16 = q_v[pl.ds(j * _LANES, _LANES)]
            e0 = jnp.maximum(q16, 0.0)
            e1 = jnp.maximum(-q16, 0.0)
            t0_v[pl.ds(j * _LANES, _LANES)] = e0 / jnp.maximum(e0, 1.0)
            t1_v[pl.ds(j * _LANES, _LANES)] = e1 / jnp.maximum(e1, 1.0)
            for r in range(_LANES):
                an16_v[r, pl.ds(j * _LANES, _LANES)] = zeros

        nch = jnp.where(is_last, last // _LANES, chunk // _LANES)

        @plsc.parallel_loop(0, nch, unroll=8)
        def _(i):
            offs = i * _LANES
            idx16 = seg_v[pl.ds(offs, _LANES)]
            g0 = g0_v[pl.ds(offs, _LANES)]
            g1 = g1_v[pl.ds(offs, _LANES)]
            eh0 = plsc.load_gather(t0_v, [idx16])
            eh1 = plsc.load_gather(t1_v, [idx16])
            d = (eh0 * g0 + eh1 * g1) * inv_sqrt_d
            a = _softplus16(d)
            a_v[pl.ds(offs, _LANES)] = a
            plsc.addupdate_scatter(an16_v, [iota, idx16], a)

        @plsc.parallel_loop(0, b // _LANES, unroll=2)
        def _(j):
            acc = an16_v[0, pl.ds(j * _LANES, _LANES)]
            for r in range(1, _LANES):
                acc = acc + an16_v[r, pl.ds(j * _LANES, _LANES)]
            part_v[pl.ds(j * _LANES, _LANES)] = acc

        @pl.when(jnp.logical_not(is_last))
        def _():
            pltpu.sync_copy(a_v, a_hbm.at[pl.ds(base, chunk)])

        @pl.when(is_last)
        def _():
            pltpu.sync_copy(a_v.at[pl.ds(0, last)],
                            a_hbm.at[pl.ds(base, last)])

        pltpu.sync_copy(part_v, part_hbm.at[wid])

    return sc1


def _make_sc2(n, b, nw, nc):
    chunk, last = _worker_extent(n, nw)
    mesh = plsc.VectorSubcoreMesh(core_axis_name="c", subcore_axis_name="s")

    @functools.partial(
        pl.kernel,
        mesh=mesh,
        compiler_params=pltpu.CompilerParams(needs_layout_passes=False),
        out_type=[
            jax.ShapeDtypeStruct((n,), jnp.float32),
            jax.ShapeDtypeStruct((n,), jnp.float32),
        ],
        scratch_types=[
            pltpu.VMEM((b,), jnp.float32),
            pltpu.VMEM((b,), jnp.float32),
            pltpu.VMEM((b,), jnp.float32),
            pltpu.VMEM((nw, b), jnp.float32),
            pltpu.VMEM((b,), jnp.float32),
            pltpu.VMEM((chunk,), jnp.int32),
            pltpu.VMEM((chunk,), jnp.float32),
            pltpu.VMEM((chunk,), jnp.float32),
            pltpu.VMEM((chunk,), jnp.float32),
        ],
    )
    def sc2(q_hbm, seg_hbm, a_hbm, part_hbm, u0_hbm, u1_hbm,
            q_v, e0_v, e1_v, pall_v, an_v, seg_v, a_v, u0_v, u1_v):
        wid = lax.axis_index("s") * nc + lax.axis_index("c")
        base = wid * chunk
        is_last = wid == nw - 1
        pltpu.sync_copy(q_hbm, q_v)
        pltpu.sync_copy(part_hbm, pall_v)

        @pl.when(jnp.logical_not(is_last))
        def _():
            pltpu.sync_copy(seg_hbm.at[pl.ds(base, chunk)], seg_v)
            pltpu.sync_copy(a_hbm.at[pl.ds(base, chunk)], a_v)

        @pl.when(is_last)
        def _():
            pltpu.sync_copy(seg_hbm.at[pl.ds(base, last)],
                            seg_v.at[pl.ds(0, last)])
            pltpu.sync_copy(a_hbm.at[pl.ds(base, last)],
                            a_v.at[pl.ds(0, last)])

        @plsc.parallel_loop(0, b // _LANES, unroll=2)
        def _(j):
            q16 = q_v[pl.ds(j * _LANES, _LANES)]
            e0_v[pl.ds(j * _LANES, _LANES)] = jnp.maximum(q16, 0.0)
            e1_v[pl.ds(j * _LANES, _LANES)] = jnp.maximum(-q16, 0.0)
            acc = pall_v[0, pl.ds(j * _LANES, _LANES)]
            for r in range(1, nw):
                acc = acc + pall_v[r, pl.ds(j * _LANES, _LANES)]
            an_v[pl.ds(j * _LANES, _LANES)] = acc + _EPS

        nch = jnp.where(is_last, last // _LANES, chunk // _LANES)

        @plsc.parallel_loop(0, nch, unroll=8)
        def _(i):
            offs = i * _LANES
            idx16 = seg_v[pl.ds(offs, _LANES)]
            a16 = a_v[pl.ds(offs, _LANES)]
            av = plsc.load_gather(an_v, [idx16])
            c = a16 / av
            u0_v[pl.ds(offs, _LANES)] = c * plsc.load_gather(e0_v, [idx16])
            u1_v[pl.ds(offs, _LANES)] = c * plsc.load_gather(e1_v, [idx16])

        @pl.when(jnp.logical_not(is_last))
        def _():
            pltpu.sync_copy(u0_v, u0_hbm.at[pl.ds(base, chunk)])
            pltpu.sync_copy(u1_v, u1_hbm.at[pl.ds(base, chunk)])

        @pl.when(is_last)
        def _():
            pltpu.sync_copy(u0_v.at[pl.ds(0, last)],
                            u0_hbm.at[pl.ds(base, last)])
            pltpu.sync_copy(u1_v.at[pl.ds(0, last)],
                            u1_hbm.at[pl.ds(base, last)])

    return sc2


def kernel(atom_embedding, Q, batch_seg, Wq, bq, Wk, Wv, W1, W2, Wout):
    n, d = atom_embedding.shape
    b = Q.shape[0]
    inv_sqrt_d = 1.0 / float(d) ** 0.5

    info = plsc.get_sparse_core_info()
    nc, ns = info.num_cores, info.num_subcores
    nw = nc * ns

    tile = 8192
    nb = pl.cdiv(n, tile)

    seg32 = batch_seg.astype(jnp.int32)
    bq2 = bq.reshape(1, d)

    g0, g1 = pl.pallas_call(
        _tc1_body,
        grid=(nb,),
        in_specs=[
            pl.BlockSpec((tile, d), lambda i: (i, 0)),
            pl.BlockSpec((d, d), lambda i: (0, 0)),
            pl.BlockSpec((2, d), lambda i: (0, 0)),
            pl.BlockSpec((1, d), lambda i: (0, 0)),
        ],
        out_specs=[
            pl.BlockSpec((tile,), lambda i: (i,)),
            pl.BlockSpec((tile,), lambda i: (i,)),
        ],
        out_shape=[
            jax.ShapeDtypeStruct((n,), jnp.float32),
            jax.ShapeDtypeStruct((n,), jnp.float32),
        ],
    )(atom_embedding, Wq, Wk, bq2)

    a_arr, parts = _make_sc1(n, b, nw, nc, inv_sqrt_d)(Q, seg32, g0, g1)

    u0, u1 = _make_sc2(n, b, nw, nc)(Q, seg32, a_arr, parts)

    out = pl.pallas_call(
        _tc2_body,
        grid=(nb,),
        in_specs=[
            pl.BlockSpec((tile,), lambda i: (i,)),
            pl.BlockSpec((tile,), lambda i: (i,)),
            pl.BlockSpec((2, d), lambda i: (0, 0)),
            pl.BlockSpec((d, d), lambda i: (0, 0)),
            pl.BlockSpec((d, d), lambda i: (0, 0)),
            pl.BlockSpec((d, d), lambda i: (0, 0)),
        ],
        out_specs=pl.BlockSpec((tile, d), lambda i: (i, 0)),
        out_shape=jax.ShapeDtypeStruct((n, d), jnp.float32),
    )(u0, u1, Wv, W1, W2, Wout)
    return out


# speedup vs baseline: 1.0059x; 1.0059x over previous
import functools

import jax
import jax.numpy as jnp
from jax import lax
from jax.experimental import pallas as pl
from jax.experimental.pallas import tpu as pltpu
from jax.experimental.pallas import tpu_sc as plsc

_EPS = 1e-8
_LANES = 16

_LOG1P_C = (
    0.9999999978515015, -0.49999970549016043, 0.3333233204385284,
    -0.24985166951419607, 0.1988030687513739, -0.1607517519300022,
    0.12363706209802593, -0.0818373380867524, 0.041082179468388526,
    -0.013268948579038478, 0.0020109670905085796,
)


def _silu(x):
    return x * (0.5 * jnp.tanh(0.5 * x) + 0.5)


def _tc1_body(a_ref, wq_ref, wk_ref, bq_ref, g0_ref, g1_ref):
    mt = lax.dot_general(wk_ref[...], wq_ref[...], (((1,), (1,)), ((), ())),
                         preferred_element_type=jnp.float32)
    bk = lax.dot_general(wk_ref[...], bq_ref[...], (((1,), (1,)), ((), ())),
                         preferred_element_type=jnp.float32)
    gt = lax.dot_general(mt, a_ref[...], (((1,), (1,)), ((), ())),
                         preferred_element_type=jnp.float32) + bk
    t = gt.shape[1]
    g0_ref[...] = gt[0:1, :].reshape(t)
    g1_ref[...] = gt[1:2, :].reshape(t)


def _silu_bf(x):
    xb = x.astype(jnp.bfloat16)
    return xb * (0.5 * jnp.tanh(0.5 * xb) + 0.5)


def _tc2_body(u0_ref, u1_ref, wv_ref, w1_ref, w2_ref, wo_ref, o_ref):
    bf = jnp.bfloat16
    t = u0_ref.shape[0]
    ut = jnp.concatenate([u0_ref[...].reshape(1, t),
                          u1_ref[...].reshape(1, t)], axis=0)
    x = lax.dot_general(ut, wv_ref[...], (((0,), (0,)), ((), ())),
                        preferred_element_type=jnp.float32)
    s = jnp.dot(_silu_bf(x), w1_ref[...].astype(bf),
                preferred_element_type=jnp.float32)
    y = x + jnp.dot(_silu_bf(s), w2_ref[...].astype(bf),
                    preferred_element_type=jnp.float32)
    o_ref[...] = jnp.dot(_silu_bf(y), wo_ref[...].astype(bf),
                         preferred_element_type=jnp.float32)


def _softplus16(d):
    t = jnp.exp(-jnp.abs(d))
    p = jnp.full((_LANES,), _LOG1P_C[-1], jnp.float32)
    for c in reversed(_LOG1P_C[:-1]):
        p = p * t + c
    return jnp.maximum(d, 0.0) + p * t


def _worker_extent(n, nw):
    chunk = ((n + nw - 1) // nw + _LANES - 1) // _LANES * _LANES
    last = n - (nw - 1) * chunk
    assert last > 0 and last % _LANES == 0 and chunk % _LANES == 0
    return chunk, last


def _make_sc1(n, b, nw, nc, inv_sqrt_d):
    chunk, last = _worker_extent(n, nw)
    mesh = plsc.VectorSubcoreMesh(core_axis_name="c", subcore_axis_name="s")

    @functools.partial(
        pl.kernel,
        mesh=mesh,
        compiler_params=pltpu.CompilerParams(needs_layout_passes=False),
        out_type=[
            jax.ShapeDtypeStruct((n,), jnp.float32),
            jax.ShapeDtypeStruct((nw, b), jnp.float32),
        ],
        scratch_types=[
            pltpu.VMEM((b,), jnp.float32),
            pltpu.VMEM((b,), jnp.float32),
            pltpu.VMEM((b,), jnp.float32),
            pltpu.VMEM((chunk,), jnp.int32),
            pltpu.VMEM((chunk,), jnp.float32),
            pltpu.VMEM((chunk,), jnp.float32),
            pltpu.VMEM((chunk,), jnp.float32),
            pltpu.VMEM((_LANES, b), jnp.float32),
            pltpu.VMEM((b,), jnp.float32),
        ],
    )
    def sc1(q_hbm, seg_hbm, g0_hbm, g1_hbm, a_hbm, part_hbm,
            q_v, t0_v, t1_v, seg_v, g0_v, g1_v, a_v, an16_v, part_v):
        wid = lax.axis_index("s") * nc + lax.axis_index("c")
        base = wid * chunk
        is_last = wid == nw - 1
        pltpu.sync_copy(q_hbm, q_v)

        @pl.when(jnp.logical_not(is_last))
        def _():
            pltpu.sync_copy(seg_hbm.at[pl.ds(base, chunk)], seg_v)
            pltpu.sync_copy(g0_hbm.at[pl.ds(base, chunk)], g0_v)
            pltpu.sync_copy(g1_hbm.at[pl.ds(base, chunk)], g1_v)

        @pl.when(is_last)
        def _():
            pltpu.sync_copy(seg_hbm.at[pl.ds(base, last)],
                            seg_v.at[pl.ds(0, last)])
            pltpu.sync_copy(g0_hbm.at[pl.ds(base, last)],
                            g0_v.at[pl.ds(0, last)])
            pltpu.sync_copy(g1_hbm.at[pl.ds(base, last)],
                            g1_v.at[pl.ds(0, last)])

        iota = lax.iota(jnp.int32, _LANES)
        zeros = jnp.zeros((_LANES,), jnp.float32)

        @plsc.parallel_loop(0, b // _LANES, unroll=2)
        def _(j):
            q16 = q_v[pl.ds(j * _LANES, _LANES)]
            e0 = jnp.maximum(q16, 0.0)
            e1 = jnp.maximum(-q16, 0.0)
            t0_v[pl.ds(j * _LANES, _LANES)] = e0 / jnp.maximum(e0, 1.0)
            t1_v[pl.ds(j * _LANES, _LANES)] = e1 / jnp.maximum(e1, 1.0)
            for r in range(_LANES):
                an16_v[r, pl.ds(j * _LANES, _LANES)] = zeros

        nch = jnp.where(is_last, last // _LANES, chunk // _LANES)

        @plsc.parallel_loop(0, nch, unroll=4)
        def _(i):
            offs = i * _LANES
            idx16 = seg_v[pl.ds(offs, _LANES)]
            g0 = g0_v[pl.ds(offs, _LANES)]
            g1 = g1_v[pl.ds(offs, _LANES)]
            eh0 = plsc.load_gather(t0_v, [idx16])
            eh1 = plsc.load_gather(t1_v, [idx16])
            d = (eh0 * g0 + eh1 * g1) * inv_sqrt_d
            a = _softplus16(d)
            a_v[pl.ds(offs, _LANES)] = a
            plsc.addupdate_scatter(an16_v, [iota, idx16], a)

        @plsc.parallel_loop(0, b // _LANES, unroll=2)
        def _(j):
            acc = an16_v[0, pl.ds(j * _LANES, _LANES)]
            for r in range(1, _LANES):
                acc = acc + an16_v[r, pl.ds(j * _LANES, _LANES)]
            part_v[pl.ds(j * _LANES, _LANES)] = acc

        @pl.when(jnp.logical_not(is_last))
        def _():
            pltpu.sync_copy(a_v, a_hbm.at[pl.ds(base, chunk)])

        @pl.when(is_last)
        def _():
            pltpu.sync_copy(a_v.at[pl.ds(0, last)],
                            a_hbm.at[pl.ds(base, last)])

        pltpu.sync_copy(part_v, part_hbm.at[wid])

    return sc1


def _make_sc2(n, b, nw, nc):
    chunk, last = _worker_extent(n, nw)
    mesh = plsc.VectorSubcoreMesh(core_axis_name="c", subcore_axis_name="s")

    @functools.partial(
        pl.kernel,
        mesh=mesh,
        compiler_params=pltpu.CompilerParams(needs_layout_passes=False),
        out_type=[
            jax.ShapeDtypeStruct((n,), jnp.float32),
            jax.ShapeDtypeStruct((n,), jnp.float32),
        ],
        scratch_types=[
            pltpu.VMEM((b,), jnp.float32),
            pltpu.VMEM((b,), jnp.float32),
            pltpu.VMEM((b,), jnp.float32),
            pltpu.VMEM((nw, b), jnp.float32),
            pltpu.VMEM((b,), jnp.float32),
            pltpu.VMEM((chunk,), jnp.int32),
            pltpu.VMEM((chunk,), jnp.float32),
            pltpu.VMEM((chunk,), jnp.float32),
            pltpu.VMEM((chunk,), jnp.float32),
        ],
    )
    def sc2(q_hbm, seg_hbm, a_hbm, part_hbm, u0_hbm, u1_hbm,
            q_v, e0_v, e1_v, pall_v, an_v, seg_v, a_v, u0_v, u1_v):
        wid = lax.axis_index("s") * nc + lax.axis_index("c")
        base = wid * chunk
        is_last = wid == nw - 1
        pltpu.sync_copy(q_hbm, q_v)
        pltpu.sync_copy(part_hbm, pall_v)

        @pl.when(jnp.logical_not(is_last))
        def _():
            pltpu.sync_copy(seg_hbm.at[pl.ds(base, chunk)], seg_v)
            pltpu.sync_copy(a_hbm.at[pl.ds(base, chunk)], a_v)

        @pl.when(is_last)
        def _():
            pltpu.sync_copy(seg_hbm.at[pl.ds(base, last)],
                            seg_v.at[pl.ds(0, last)])
            pltpu.sync_copy(a_hbm.at[pl.ds(base, last)],
                            a_v.at[pl.ds(0, last)])

        @plsc.parallel_loop(0, b // _LANES, unroll=2)
        def _(j):
            q16 = q_v[pl.ds(j * _LANES, _LANES)]
            e0_v[pl.ds(j * _LANES, _LANES)] = jnp.maximum(q16, 0.0)
            e1_v[pl.ds(j * _LANES, _LANES)] = jnp.maximum(-q16, 0.0)
            acc = pall_v[0, pl.ds(j * _LANES, _LANES)]
            for r in range(1, nw):
                acc = acc + pall_v[r, pl.ds(j * _LANES, _LANES)]
            an_v[pl.ds(j * _LANES, _LANES)] = acc + _EPS

        nch = jnp.where(is_last, last // _LANES, chunk // _LANES)

        @plsc.parallel_loop(0, nch, unroll=4)
        def _(i):
            offs = i * _LANES
            idx16 = seg_v[pl.ds(offs, _LANES)]
            a16 = a_v[pl.ds(offs, _LANES)]
            av = plsc.load_gather(an_v, [idx16])
            c = a16 / av
            u0_v[pl.ds(offs, _LANES)] = c * plsc.load_gather(e0_v, [idx16])
            u1_v[pl.ds(offs, _LANES)] = c * plsc.load_gather(e1_v, [idx16])

        @pl.when(jnp.logical_not(is_last))
        def _():
            pltpu.sync_copy(u0_v, u0_hbm.at[pl.ds(base, chunk)])
            pltpu.sync_copy(u1_v, u1_hbm.at[pl.ds(base, chunk)])

        @pl.when(is_last)
        def _():
            pltpu.sync_copy(u0_v.at[pl.ds(0, last)],
                            u0_hbm.at[pl.ds(base, last)])
            pltpu.sync_copy(u1_v.at[pl.ds(0, last)],
                            u1_hbm.at[pl.ds(base, last)])

    return sc2


def kernel(atom_embedding, Q, batch_seg, Wq, bq, Wk, Wv, W1, W2, Wout):
    n, d = atom_embedding.shape
    b = Q.shape[0]
    inv_sqrt_d = 1.0 / float(d) ** 0.5

    info = plsc.get_sparse_core_info()
    nc, ns = info.num_cores, info.num_subcores
    nw = nc * ns

    tile = 8192
    nb = pl.cdiv(n, tile)

    seg32 = batch_seg.astype(jnp.int32)
    bq2 = bq.reshape(1, d)

    g0, g1 = pl.pallas_call(
        _tc1_body,
        grid=(nb,),
        in_specs=[
            pl.BlockSpec((tile, d), lambda i: (i, 0)),
            pl.BlockSpec((d, d), lambda i: (0, 0)),
            pl.BlockSpec((2, d), lambda i: (0, 0)),
            pl.BlockSpec((1, d), lambda i: (0, 0)),
        ],
        out_specs=[
            pl.BlockSpec((tile,), lambda i: (i,)),
            pl.BlockSpec((tile,), lambda i: (i,)),
        ],
        out_shape=[
            jax.ShapeDtypeStruct((n,), jnp.float32),
            jax.ShapeDtypeStruct((n,), jnp.float32),
        ],
    )(atom_embedding, Wq, Wk, bq2)

    a_arr, parts = _make_sc1(n, b, nw, nc, inv_sqrt_d)(Q, seg32, g0, g1)

    u0, u1 = _make_sc2(n, b, nw, nc)(Q, seg32, a_arr, parts)

    out = pl.pallas_call(
        _tc2_body,
        grid=(nb,),
        in_specs=[
            pl.BlockSpec((tile,), lambda i: (i,)),
            pl.BlockSpec((tile,), lambda i: (i,)),
            pl.BlockSpec((2, d), lambda i: (0, 0)),
            pl.BlockSpec((d, d), lambda i: (0, 0)),
            pl.BlockSpec((d, d), lambda i: (0, 0)),
            pl.BlockSpec((d, d), lambda i: (0, 0)),
        ],
        out_specs=pl.BlockSpec((tile, d), lambda i: (i, 0)),
        out_shape=jax.ShapeDtypeStruct((n, d), jnp.float32),
    )(u0, u1, Wv, W1, W2, Wout)
    return out
